# baseline (speedup 1.0000x reference)
import functools

import jax
import jax.numpy as jnp
from jax import lax
from jax.experimental import pallas as pl
from jax.experimental.pallas import tpu as pltpu
from jax.experimental.pallas import tpu_sc as plsc

NB = 10000
B = 4
HIDDEN = 64
NUM_BLOCKS = 4
S_MAX = 1.0
DATA_DIM = 2 * NB
E = 160000
EPAD = 163840
NROWS = EPAD // 128
NBPAD = 10240
RPS = NBPAD // 16


def _mp_body(CRO, SR, X0, X1, I0, I1, srcr, dstr, A0, A1,
             idx_s, idx_d, rows, acc, gsem, ssem):
    c = lax.axis_index("c")
    s = lax.axis_index("s")
    r0 = s * RPS

    @pl.when(c == 0)
    def _():
        pltpu.sync_copy(I0.at[pl.ds(r0, RPS)], acc.at[pl.ds(r0, RPS)])

    @pl.when(c == 1)
    def _():
        pltpu.sync_copy(I1.at[pl.ds(r0, RPS)], acc.at[pl.ds(r0, RPS)])

    plsc.subcore_barrier()

    row0 = c * CRO + s * SR
    pltpu.sync_copy(srcr.at[pl.ds(row0, SR)], idx_s)
    pltpu.sync_copy(dstr.at[pl.ds(row0, SR)], idx_d)

    def start_gather(j, b):
        @pl.when(c == 0)
        def _():
            pltpu.async_copy(X0.at[idx_s.at[j]], rows.at[b], gsem.at[b])

        @pl.when(c == 1)
        def _():
            pltpu.async_copy(X1.at[idx_s.at[j]], rows.at[b], gsem.at[b])

    def wait_gather(b):
        pltpu.make_async_copy(X0.at[idx_s.at[0]], rows.at[b],
                              gsem.at[b]).wait()

    def start_scatter(j, b):
        pltpu.async_copy(rows.at[b], acc.at[idx_d.at[j]], ssem.at[b],
                         add=True)

    def wait_scatter(b):
        pltpu.make_async_copy(rows.at[b], acc.at[idx_d.at[0]],
                              ssem.at[b]).wait()

    start_gather(0, 0)

    def outer(g, carry):
        for b in range(2):
            j = g * 2 + b
            wait_gather(b)

            @pl.when(j >= 1)
            def _():
                wait_scatter(1 - b)

            @pl.when(j + 1 < SR)
            def _():
                start_gather(j + 1, 1 - b)

            start_scatter(j, b)
        return carry

    lax.fori_loop(0, SR // 2, outer, 0)
    wait_scatter((SR - 1) % 2)
    plsc.subcore_barrier()

    @pl.when(c == 0)
    def _():
        pltpu.sync_copy(acc.at[pl.ds(r0, RPS)], A0.at[pl.ds(r0, RPS)])

    @pl.when(c == 1)
    def _():
        pltpu.sync_copy(acc.at[pl.ds(r0, RPS)], A1.at[pl.ds(r0, RPS)])


@functools.lru_cache(maxsize=None)
def _make_mp(W=128):
    SR = NROWS // 32
    CRO = NROWS // 2
    f32 = jnp.float32
    return pl.kernel(
        functools.partial(_mp_body, CRO, SR),
        mesh=plsc.VectorSubcoreMesh(core_axis_name="c", subcore_axis_name="s"),
        out_type=[jax.ShapeDtypeStruct((NBPAD, W), f32),
                  jax.ShapeDtypeStruct((NBPAD, W), f32)],
        scratch_types=[
            pltpu.VMEM((SR, 128), jnp.int32),
            pltpu.VMEM((SR, 128), jnp.int32),
            pltpu.VMEM((2, 128, W), f32),
            pltpu.VMEM_SHARED((NBPAD, W), f32),
            pltpu.SemaphoreType.DMA((2,)),
            pltpu.SemaphoreType.DMA((2,)),
        ],
    )


def _silu(x):
    return x * jax.nn.sigmoid(x)


def _padn(x, w=128):
    out = jnp.zeros((NBPAD, w), x.dtype)
    return lax.dynamic_update_slice(out, x, (0, 0))


def _forward(z, edge_index, params, perms, mp):
    f32 = jnp.float32
    ei = edge_index.astype(jnp.int32)
    pad_s = jnp.zeros((EPAD - E,), jnp.int32)
    pad_d = jnp.full((EPAD - E,), NB, jnp.int32)
    srcr = jnp.concatenate([ei[0], pad_s]).reshape(NROWS, 128)
    dstr = jnp.concatenate([ei[1], pad_d]).reshape(NROWS, 128)

    ones = jnp.ones((NBPAD, 128), f32)
    zeros = jnp.zeros((NBPAD, 128), f32)
    d0, d1 = mp(ones, ones, ones, zeros, srcr, dstr)
    deg = (d0[:, 0] + d1[:, 0])[:NB]
    dinv = lax.rsqrt(deg)

    base = (jnp.arange(NB) % 2).astype(f32)
    y = z
    for i in range(NUM_BLOCKS):
        mask = base if i % 2 == 0 else 1.0 - base
        bp = params["blocks"][i]
        X = y[:, perms[i]].reshape(B, NB, 2).transpose(1, 0, 2)
        m = mask[:, None, None]
        Xm = X * m

        flat = Xm.reshape(NB, B * 2) * dinv[:, None]
        Xs1 = _padn(flat)
        a0, a1 = mp(Xs1, Xs1, Xs1, zeros, srcr, dstr)
        XP1 = ((a0 + a1)[:NB, :B * 2] * dinv[:, None]).reshape(NB, B, 2)
        H = _silu(Xm @ bp["conv1"][0]["W"] + bp["conv1"][0]["b"]
                  + XP1 @ bp["conv1"][1]["W"] + bp["conv1"][1]["b"])

        Xs2 = (H * dinv[:, None, None]).reshape(NB, B * HIDDEN)
        x0h, x1h = _padn(Xs2[:, :128]), _padn(Xs2[:, 128:])
        a0, a1 = mp(x0h, x0h, x0h, zeros, srcr, dstr)
        b0, b1 = mp(x1h, x1h, x1h, zeros, srcr, dstr)
        XP2 = (jnp.concatenate([(a0 + a1)[:NB], (b0 + b1)[:NB]], axis=1)
               * dinv[:, None]).reshape(NB, B, HIDDEN)
        H = _silu(H @ bp["conv2"][0]["W"] + bp["conv2"][0]["b"]
                  + XP2 @ bp["conv2"][1]["W"] + bp["conv2"][1]["b"])

        H = _silu(H @ bp["head"][0]["W"] + bp["head"][0]["b"])
        out = H @ bp["head"][1]["W"] + bp["head"][1]["b"]
        log_s = S_MAX * jnp.tanh(out[..., :2])
        bb = out[..., 2:]
        inv = 1.0 - m
        Yn = Xm + inv * (jnp.exp(log_s) * (X * inv) + bb)
        y = Yn.transpose(1, 0, 2).reshape(B, DATA_DIM)
    return y


def kernel(z, edge_index, params, perms):
    return _forward(z, edge_index, params, perms, _make_mp())


# speedup vs baseline: 1.0198x; 1.0198x over previous
import functools

import jax
import jax.numpy as jnp
from jax import lax
from jax.experimental import pallas as pl
from jax.experimental.pallas import tpu as pltpu
from jax.experimental.pallas import tpu_sc as plsc

NB = 10000
B = 4
HIDDEN = 64
NUM_BLOCKS = 4
S_MAX = 1.0
DATA_DIM = 2 * NB
E = 160000
EPAD = 163840
NROWS = EPAD // 128
NBPAD = 10240
RPS = NBPAD // 16


def _mp_body(CRO, SR, X0, X1, I0, I1, srcr, dstr, A0, A1,
             idx_s, idx_d, rows, acc, gsem, ssem):
    c = lax.axis_index("c")
    s = lax.axis_index("s")
    r0 = s * RPS

    @pl.when(c == 0)
    def _():
        pltpu.sync_copy(I0.at[pl.ds(r0, RPS)], acc.at[pl.ds(r0, RPS)])

    @pl.when(c == 1)
    def _():
        pltpu.sync_copy(I1.at[pl.ds(r0, RPS)], acc.at[pl.ds(r0, RPS)])

    plsc.subcore_barrier()

    row0 = c * CRO + s * SR
    pltpu.sync_copy(srcr.at[pl.ds(row0, SR)], idx_s)
    pltpu.sync_copy(dstr.at[pl.ds(row0, SR)], idx_d)

    def start_gather(j, b):
        @pl.when(c == 0)
        def _():
            pltpu.async_copy(X0.at[idx_s.at[j]], rows.at[b], gsem.at[b])

        @pl.when(c == 1)
        def _():
            pltpu.async_copy(X1.at[idx_s.at[j]], rows.at[b], gsem.at[b])

    def wait_gather(b):
        pltpu.make_async_copy(X0.at[idx_s.at[0]], rows.at[b],
                              gsem.at[b]).wait()

    def start_scatter(j, b):
        pltpu.async_copy(rows.at[b], acc.at[idx_d.at[j]], ssem.at[b],
                         add=True)

    def wait_scatter(b):
        pltpu.make_async_copy(rows.at[b], acc.at[idx_d.at[0]],
                              ssem.at[b]).wait()

    start_gather(0, 0)

    def outer(g, carry):
        for b in range(2):
            j = g * 2 + b
            wait_gather(b)

            @pl.when(j >= 1)
            def _():
                wait_scatter(1 - b)

            @pl.when(j + 1 < SR)
            def _():
                start_gather(j + 1, 1 - b)

            start_scatter(j, b)
        return carry

    lax.fori_loop(0, SR // 2, outer, 0)
    wait_scatter((SR - 1) % 2)
    plsc.subcore_barrier()

    @pl.when(c == 0)
    def _():
        pltpu.sync_copy(acc.at[pl.ds(r0, RPS)], A0.at[pl.ds(r0, RPS)])

    @pl.when(c == 1)
    def _():
        pltpu.sync_copy(acc.at[pl.ds(r0, RPS)], A1.at[pl.ds(r0, RPS)])


@functools.lru_cache(maxsize=None)
def _make_mp(W=128):
    SR = NROWS // 32
    CRO = NROWS // 2
    f32 = jnp.float32
    return pl.kernel(
        functools.partial(_mp_body, CRO, SR),
        mesh=plsc.VectorSubcoreMesh(core_axis_name="c", subcore_axis_name="s"),
        out_type=[jax.ShapeDtypeStruct((NBPAD, W), f32),
                  jax.ShapeDtypeStruct((NBPAD, W), f32)],
        scratch_types=[
            pltpu.VMEM((SR, 128), jnp.int32),
            pltpu.VMEM((SR, 128), jnp.int32),
            pltpu.VMEM((2, 128, W), f32),
            pltpu.VMEM_SHARED((NBPAD, W), f32),
            pltpu.SemaphoreType.DMA((2,)),
            pltpu.SemaphoreType.DMA((2,)),
        ],
    )


def _silu(x):
    return x * jax.nn.sigmoid(x)


_GRID = 10
_RB = NB // _GRID


def _row_mask(i_par):
    pid = pl.program_id(0)
    rows = lax.broadcasted_iota(jnp.int32, (_RB, 1), 0) + pid * _RB
    return ((rows + i_par) % 2).astype(jnp.float32)


def _prep_body(i_par, xg, dinvc, xm, xs):
    m = _row_mask(i_par)
    xmv = xg[...] * m
    xm[...] = xmv
    xs[...] = xmv * dinvc[...]


def _d1_body(xm, a0, a1, dinvc, w0, w1, bt, h, xs2):
    dv = dinvc[...]
    xp8 = (a0[...][:, :8] + a1[...][:, :8]) * dv
    hh = _silu(xm[...] @ w0[...] + xp8 @ w1[...] + bt[...])
    h[...] = hh
    xs2[...] = hh * dv


def _d2_body(i_par, h, a0, a1, b0, b1, xg, dinvc,
             w20, w21a, w21b, bt2, wh0, bth, wh1, btl, y):
    dv = dinvc[...]
    xpa = (a0[...] + a1[...]) * dv
    xpb = (b0[...] + b1[...]) * dv
    h2 = _silu(h[...] @ w20[...] + xpa @ w21a[...] + xpb @ w21b[...]
               + bt2[...])
    h3 = _silu(h2 @ wh0[...] + bth[...])
    out = h3 @ wh1[...] + btl[...]
    m = _row_mask(i_par)
    x = xg[...]
    cols = []
    for b in range(B):
        ls = out[:, 4 * b:4 * b + 2]
        bb = out[:, 4 * b + 2:4 * b + 4]
        xb = x[:, 2 * b:2 * b + 2]
        cols.append(m * xb + (1.0 - m)
                    * (jnp.exp(S_MAX * jnp.tanh(ls)) * xb + bb))
    y[...] = jnp.concatenate(cols, axis=1)


def _rspec(w):
    return pl.BlockSpec((_RB, w), lambda g: (g, 0))


def _cspec(shape):
    return pl.BlockSpec(shape, lambda g: (0, 0))


def _tc_prep(i_par, xg8, dinvc):
    f32 = jnp.float32
    return pl.pallas_call(
        functools.partial(_prep_body, i_par),
        grid=(_GRID,),
        in_specs=[_rspec(8), _rspec(1)],
        out_specs=[_rspec(8), _rspec(8)],
        out_shape=[jax.ShapeDtypeStruct((NB, 8), f32)] * 2,
    )(xg8, dinvc)


def _tc_d1(xm8, a0, a1, dinvc, w0, w1, bt):
    f32 = jnp.float32
    return pl.pallas_call(
        _d1_body,
        grid=(_GRID,),
        in_specs=[_rspec(8), _rspec(128), _rspec(128), _rspec(1),
                  _cspec((8, 256)), _cspec((8, 256)), _cspec((1, 256))],
        out_specs=[_rspec(256), _rspec(256)],
        out_shape=[jax.ShapeDtypeStruct((NB, 256), f32)] * 2,
    )(xm8, a0, a1, dinvc, w0, w1, bt)


def _tc_d2(i_par, h, a0, a1, b0, b1, xg8, dinvc, ws):
    f32 = jnp.float32
    return pl.pallas_call(
        functools.partial(_d2_body, i_par),
        grid=(_GRID,),
        in_specs=[_rspec(256), _rspec(128), _rspec(128), _rspec(128),
                  _rspec(128), _rspec(8), _rspec(1),
                  _cspec((256, 256)), _cspec((128, 256)),
                  _cspec((128, 256)), _cspec((1, 256)),
                  _cspec((256, 256)), _cspec((1, 256)),
                  _cspec((256, 16)), _cspec((1, 16))],
        out_specs=_rspec(8),
        out_shape=jax.ShapeDtypeStruct((NB, 8), f32),
    )(h, a0, a1, b0, b1, xg8, dinvc, *ws)


def _padn(x, w=128):
    out = jnp.zeros((NBPAD, w), x.dtype)
    return lax.dynamic_update_slice(out, x, (0, 0))


def _forward(z, edge_index, params, perms, mp):
    f32 = jnp.float32
    ei = edge_index.astype(jnp.int32)
    pad_s = jnp.zeros((EPAD - E,), jnp.int32)
    pad_d = jnp.full((EPAD - E,), NB, jnp.int32)
    srcr = jnp.concatenate([ei[0], pad_s]).reshape(NROWS, 128)
    dstr = jnp.concatenate([ei[1], pad_d]).reshape(NROWS, 128)

    ones = jnp.ones((NBPAD, 128), f32)
    zeros = jnp.zeros((NBPAD, 128), f32)
    d0, d1 = mp(ones, ones, ones, zeros, srcr, dstr)
    deg = (d0[:, 0] + d1[:, 0])[:NB]
    dinv = lax.rsqrt(deg)

    dinvc = dinv[:, None]
    eye = jnp.eye(B, dtype=f32)

    y = z
    for i in range(NUM_BLOCKS):
        bp = params["blocks"][i]
        w10 = jnp.kron(eye, bp["conv1"][0]["W"])
        w11 = jnp.kron(eye, bp["conv1"][1]["W"])
        b1t = jnp.tile(bp["conv1"][0]["b"] + bp["conv1"][1]["b"], B)[None]
        w20 = jnp.kron(eye, bp["conv2"][0]["W"])
        w21 = jnp.kron(eye, bp["conv2"][1]["W"])
        b2t = jnp.tile(bp["conv2"][0]["b"] + bp["conv2"][1]["b"], B)[None]
        wh0 = jnp.kron(eye, bp["head"][0]["W"])
        bh0 = jnp.tile(bp["head"][0]["b"], B)[None]
        wh1 = jnp.kron(eye, bp["head"][1]["W"])
        bh1 = jnp.tile(bp["head"][1]["b"], B)[None]

        Xg8 = y[:, perms[i]].reshape(B, NB, 2).transpose(1, 0, 2) \
               .reshape(NB, B * 2)
        xm8, xs1u = _tc_prep(i % 2, Xg8, dinvc)
        Xs1 = _padn(xs1u)
        a0, a1 = mp(Xs1, Xs1, Xs1, zeros, srcr, dstr)
        H, Xs2 = _tc_d1(xm8, a0, a1, dinvc, w10, w11, b1t)
        x0h, x1h = _padn(Xs2[:, :128]), _padn(Xs2[:, 128:])
        c0, c1 = mp(x0h, x0h, x0h, zeros, srcr, dstr)
        e0, e1 = mp(x1h, x1h, x1h, zeros, srcr, dstr)
        y8 = _tc_d2(i % 2, H, c0, c1, e0, e1, Xg8, dinvc,
                    (w20, w21[:128], w21[128:], b2t, wh0, bh0, wh1, bh1))
        y = y8.reshape(NB, B, 2).transpose(1, 0, 2).reshape(B, DATA_DIM)
    return y


def kernel(z, edge_index, params, perms):
    return _forward(z, edge_index, params, perms, _make_mp())
